# baseline (speedup 1.0000x reference)
import functools

import jax
import jax.numpy as jnp
from jax import lax
from jax.experimental import pallas as pl
from jax.experimental.pallas import tpu as pltpu
from jax.experimental.pallas import tpu_sc as plsc

NC = 2
NS = 16
NW = NC * NS
L = 16
IDX_BLK = 128


def _sc_embed(table, responses, times, w, pos, *, B, S, D, CH):
  R = B * S
  rpw = R // NW
  nch = rpw // CH
  G = CH // IDX_BLK
  assert G % 8 == 0 and nch * CH == rpw
  mesh = plsc.VectorSubcoreMesh(core_axis_name="c", subcore_axis_name="s",
                                num_cores=NC, num_subcores=NS)

  @functools.partial(
      pl.kernel,
      out_type=jax.ShapeDtypeStruct((R, D), jnp.float32),
      mesh=mesh,
      compiler_params=pltpu.CompilerParams(use_tc_tiling_on_sc=False),
      scratch_types=[
          pltpu.VMEM((G, IDX_BLK), jnp.int32),
          pltpu.VMEM((CH,), jnp.float32),
          pltpu.VMEM((CH, D), jnp.float32),
          pltpu.VMEM((S, D), jnp.float32),
          pltpu.VMEM((D,), jnp.float32),
      ] + [pltpu.SemaphoreType.DMA] * G,
  )
  def k(table_hbm, idx_hbm, tflat_hbm, w_hbm, pos_hbm, oflat_hbm,
        idx_v, times_v, buf, pos_v, w_v, *sems):
    wid = lax.axis_index("s") * NC + lax.axis_index("c")
    base = wid * rpw
    pltpu.sync_copy(pos_hbm, pos_v)
    pltpu.sync_copy(w_hbm, w_v)
    w0 = w_v[pl.ds(0, L)]
    w1 = w_v[pl.ds(L, L)]

    def chunk(ci, _):
      row0 = base + ci * CH
      blk0 = pl.multiple_of(row0 // IDX_BLK, 8)
      pltpu.sync_copy(idx_hbm.at[pl.ds(blk0, G)], idx_v)
      pltpu.sync_copy(tflat_hbm.at[pl.ds(pl.multiple_of(row0, 8), CH)],
                      times_v)
      descs = [
          pltpu.async_copy(table_hbm.at[idx_v.at[j]],
                           buf.at[pl.ds(j * IDX_BLK, IDX_BLK)], sems[j])
          for j in range(G)
      ]
      for j in range(G):
        descs[j].wait()

        def grp(g, _):
          r0 = j * IDX_BLK + g * L
          t16 = times_v[pl.ds(r0, L)]
          for i in range(L):
            r = r0 + i
            t = t16[i]
            s = lax.rem(row0 + r, S)
            buf[r, pl.ds(0, L)] = (buf[r, pl.ds(0, L)] + t * w0
                                   + pos_v[s, pl.ds(0, L)])
            buf[r, pl.ds(L, L)] = (buf[r, pl.ds(L, L)] + t * w1
                                   + pos_v[s, pl.ds(L, L)])
          return 0

        lax.fori_loop(0, IDX_BLK // L, grp, 0)
        pltpu.sync_copy(buf.at[pl.ds(j * IDX_BLK, IDX_BLK)],
                        oflat_hbm.at[pl.ds(row0 + j * IDX_BLK, IDX_BLK)])
      return 0

    lax.fori_loop(0, nch, chunk, 0)

  out = k(table, responses.reshape(R // IDX_BLK, IDX_BLK),
          times.reshape(R), w.reshape(D), pos)
  return out.reshape(B, S, D)


def kernel(responses, solving_times, emb_response, W_time, emb_pos):
  B, S = responses.shape
  V, D = emb_response.shape
  return _sc_embed(emb_response, responses.astype(jnp.int32), solving_times,
                   W_time, emb_pos, B=B, S=S, D=D, CH=1024)


# speedup vs baseline: 1.0029x; 1.0029x over previous
import functools

import jax
import jax.numpy as jnp
from jax import lax
from jax.experimental import pallas as pl
from jax.experimental.pallas import tpu as pltpu
from jax.experimental.pallas import tpu_sc as plsc

NC = 2
NS = 16
NW = NC * NS
L = 16
IDX_BLK = 128


def _sc_embed(table, responses, times, w, pos, *, B, S, D, CH):
  R = B * S
  rpw = R // NW
  nch = rpw // CH
  G = CH // IDX_BLK
  assert G % 8 == 0 and nch * CH == rpw
  mesh = plsc.VectorSubcoreMesh(core_axis_name="c", subcore_axis_name="s",
                                num_cores=NC, num_subcores=NS)

  @functools.partial(
      pl.kernel,
      out_type=jax.ShapeDtypeStruct((R, D), jnp.float32),
      mesh=mesh,
      compiler_params=pltpu.CompilerParams(use_tc_tiling_on_sc=False),
      scratch_types=[
          pltpu.VMEM((G, IDX_BLK), jnp.int32),
          pltpu.VMEM((CH,), jnp.float32),
          pltpu.VMEM((CH, D), jnp.float32),
          pltpu.VMEM((S, D), jnp.float32),
          pltpu.VMEM((D,), jnp.float32),
      ] + [pltpu.SemaphoreType.DMA] * G,
  )
  def k(table_hbm, idx_hbm, tflat_hbm, w_hbm, pos_hbm, oflat_hbm,
        idx_v, times_v, buf, pos_v, w_v, *sems):
    wid = lax.axis_index("s") * NC + lax.axis_index("c")
    base = wid * rpw
    pltpu.sync_copy(pos_hbm, pos_v)
    pltpu.sync_copy(w_hbm, w_v)
    w0 = w_v[pl.ds(0, L)]
    w1 = w_v[pl.ds(L, L)]

    def chunk(ci, _):
      row0 = base + ci * CH
      blk0 = pl.multiple_of(row0 // IDX_BLK, 8)
      pltpu.sync_copy(idx_hbm.at[pl.ds(blk0, G)], idx_v)
      pltpu.sync_copy(tflat_hbm.at[pl.ds(pl.multiple_of(row0, 8), CH)],
                      times_v)
      descs = [
          pltpu.async_copy(table_hbm.at[idx_v.at[j]],
                           buf.at[pl.ds(j * IDX_BLK, IDX_BLK)], sems[j])
          for j in range(G)
      ]
      for j in range(G):
        descs[j].wait()

        def grp(g, _):
          r0 = j * IDX_BLK + g * L
          t16 = times_v[pl.ds(r0, L)]
          for i in range(L):
            r = r0 + i
            t = t16[i]
            s = lax.rem(row0 + r, S)
            buf[r, pl.ds(0, L)] = (buf[r, pl.ds(0, L)] + t * w0
                                   + pos_v[s, pl.ds(0, L)])
            buf[r, pl.ds(L, L)] = (buf[r, pl.ds(L, L)] + t * w1
                                   + pos_v[s, pl.ds(L, L)])
          return 0

        lax.fori_loop(0, IDX_BLK // L, grp, 0)
      pltpu.sync_copy(buf, oflat_hbm.at[pl.ds(row0, CH)])
      return 0

    lax.fori_loop(0, nch, chunk, 0)

  out = k(table, responses.reshape(R // IDX_BLK, IDX_BLK),
          times.reshape(R), w.reshape(D), pos)
  return out.reshape(B, S, D)


def kernel(responses, solving_times, emb_response, W_time, emb_pos):
  B, S = responses.shape
  V, D = emb_response.shape
  return _sc_embed(emb_response, responses.astype(jnp.int32), solving_times,
                   W_time, emb_pos, B=B, S=S, D=D, CH=1024)


# speedup vs baseline: 1.0945x; 1.0914x over previous
import functools

import jax
import jax.numpy as jnp
from jax import lax
from jax.experimental import pallas as pl
from jax.experimental.pallas import tpu as pltpu
from jax.experimental.pallas import tpu_sc as plsc

NC = 2
NS = 16
NW = NC * NS
L = 16
IDX_BLK = 128


def _sc_embed(table, responses, times, w, pos, *, B, S, D, CH):
  R = B * S
  rpw = R // NW
  nch = rpw // CH
  G = CH // IDX_BLK
  assert G % 8 == 0 and nch * CH == rpw
  mesh = plsc.VectorSubcoreMesh(core_axis_name="c", subcore_axis_name="s",
                                num_cores=NC, num_subcores=NS)

  @functools.partial(
      pl.kernel,
      out_type=jax.ShapeDtypeStruct((R, D), jnp.float32),
      mesh=mesh,
      compiler_params=pltpu.CompilerParams(use_tc_tiling_on_sc=False),
      scratch_types=[
          pltpu.VMEM((G, IDX_BLK), jnp.int32),
          pltpu.VMEM((CH,), jnp.float32),
          pltpu.VMEM((CH, D), jnp.float32),
          pltpu.VMEM((S, D), jnp.float32),
          pltpu.VMEM((D,), jnp.float32),
      ] + [pltpu.SemaphoreType.DMA] * G,
  )
  def k(table_hbm, idx_hbm, tflat_hbm, w_hbm, pos_hbm, oflat_hbm,
        idx_v, times_v, buf, pos_v, w_v, *sems):
    wid = lax.axis_index("s") * NC + lax.axis_index("c")
    base = wid * rpw
    pltpu.sync_copy(pos_hbm, pos_v)
    pltpu.sync_copy(w_hbm, w_v)
    w0 = w_v[pl.ds(0, L)]
    w1 = w_v[pl.ds(L, L)]

    def chunk(ci, _):
      row0 = base + ci * CH
      blk0 = pl.multiple_of(row0 // IDX_BLK, 8)
      pltpu.sync_copy(idx_hbm.at[pl.ds(blk0, G)], idx_v)
      pltpu.sync_copy(tflat_hbm.at[pl.ds(pl.multiple_of(row0, 8), CH)],
                      times_v)
      descs = [
          pltpu.async_copy(table_hbm.at[idx_v.at[j]],
                           buf.at[pl.ds(j * IDX_BLK, IDX_BLK)], sems[0])
          for j in range(G)
      ]
      for d in descs:
        d.wait()

      def grp(g, _):
        r0 = g * L
        t16 = times_v[pl.ds(r0, L)]
        for i in range(L):
          r = r0 + i
          t = t16[i]
          s = lax.rem(row0 + r, S)
          buf[r, pl.ds(0, L)] = (buf[r, pl.ds(0, L)] + t * w0
                                 + pos_v[s, pl.ds(0, L)])
          buf[r, pl.ds(L, L)] = (buf[r, pl.ds(L, L)] + t * w1
                                 + pos_v[s, pl.ds(L, L)])
        return 0

      lax.fori_loop(0, CH // L, grp, 0)
      pltpu.sync_copy(buf, oflat_hbm.at[pl.ds(row0, CH)])
      return 0

    lax.fori_loop(0, nch, chunk, 0)

  out = k(table, responses.reshape(R // IDX_BLK, IDX_BLK),
          times.reshape(R), w.reshape(D), pos)
  return out.reshape(B, S, D)


def kernel(responses, solving_times, emb_response, W_time, emb_pos):
  B, S = responses.shape
  V, D = emb_response.shape
  return _sc_embed(emb_response, responses.astype(jnp.int32), solving_times,
                   W_time, emb_pos, B=B, S=S, D=D, CH=1024)
